# Initial kernel scaffold; baseline (speedup 1.0000x reference)
import jax
import jax.numpy as jnp
from jax.experimental import pallas as pl


def kernel(X, A, W):
    raise NotImplementedError("write your pallas kernel here")


# speedup vs baseline: 1.7441x; 1.7441x over previous
import functools

import jax
import jax.numpy as jnp
from jax import lax
from jax.experimental import pallas as pl
from jax.experimental.pallas import tpu as pltpu
from jax.experimental.pallas import tpu_sc as plsc

N = 10000
F = 128
K = 5000
LANES = 128
NPAD = 10240
NROWS2D = NPAD // LANES
IBLK = 1024
NB = NPAD // IBLK
KPAD = 5008
RB = 4
NBLK = K // RB
NTILES = 32
BLK_PER_TILE = -(-NBLK // NTILES)
SCAT_PER_TILE = NPAD // NTILES


def _scores_body(x_ref, w_ref, o_ref):
    s = jnp.dot(x_ref[...], w_ref[...])
    rid = lax.broadcasted_iota(jnp.int32, (NPAD, 1), 0)
    o_ref[...] = jnp.where(rid < N, s, -jnp.inf)


def _scores(x_pad, w):
    return pl.pallas_call(
        _scores_body,
        out_shape=jax.ShapeDtypeStruct((NPAD, 1), jnp.float32),
    )(x_pad, w)


def _ranks_body(scol_ref, s2d_ref, o_ref):
    ib = pl.program_id(0)
    si = scol_ref[...]
    ii = ib * IBLK + lax.broadcasted_iota(jnp.int32, (IBLK, 1), 0)

    def jstep(rb, acc):
        sj = s2d_ref[pl.ds(rb, 1), :]
        jj = rb * LANES + lax.broadcasted_iota(jnp.int32, (1, LANES), 1)
        gt = sj > si
        tie = (sj == si) & (jj < ii)
        cnt = jnp.where(gt | tie, 1, 0)
        return acc + jnp.sum(cnt, axis=1, keepdims=True)

    acc = lax.fori_loop(0, NROWS2D, jstep, jnp.zeros((IBLK, 1), jnp.int32))
    o_ref[...] = acc


def _ranks(scol, s2d):
    return pl.pallas_call(
        _ranks_body,
        grid=(NB,),
        in_specs=[
            pl.BlockSpec((IBLK, 1), lambda i: (i, 0)),
            pl.BlockSpec((NROWS2D, LANES), lambda i: (0, 0)),
        ],
        out_specs=pl.BlockSpec((IBLK, 1), lambda i: (i, 0)),
        out_shape=jax.ShapeDtypeStruct((NPAD, 1), jnp.int32),
    )(scol, s2d)


def _make_scatter():
    mesh = plsc.VectorSubcoreMesh(core_axis_name="c", subcore_axis_name="s")

    @functools.partial(
        pl.kernel,
        out_type=jax.ShapeDtypeStruct((NPAD,), jnp.int32),
        mesh=mesh,
        compiler_params=pltpu.CompilerParams(needs_layout_passes=False),
        scratch_types=[
            pltpu.VMEM((SCAT_PER_TILE,), jnp.int32),
            pltpu.VMEM((SCAT_PER_TILE,), jnp.int32),
            pltpu.SemaphoreType.DMA,
        ],
    )
    def scatter_k(ranks_hbm, out_hbm, ranks_v, vals_v, sem):
        wid = lax.axis_index("s") * 2 + lax.axis_index("c")
        base = wid * SCAT_PER_TILE
        pltpu.sync_copy(ranks_hbm.at[pl.ds(base, SCAT_PER_TILE)], ranks_v)
        for g in range(SCAT_PER_TILE // 16):
            vals_v[pl.ds(g * 16, 16)] = base + g * 16 + lax.iota(jnp.int32, 16)
        for g in range(SCAT_PER_TILE // 16):
            rvec = ranks_v[pl.ds(g * 16, 16)]
            pltpu.async_copy(
                vals_v.at[pl.ds(g * 16, 16)], out_hbm.at[rvec], sem
            ).wait()

    return scatter_k


_scatter_cached = functools.cache(_make_scatter)


def _make_gather():
    mesh = plsc.VectorSubcoreMesh(core_axis_name="c", subcore_axis_name="s")

    @functools.partial(
        pl.kernel,
        out_type=(
            jax.ShapeDtypeStruct((K * F,), jnp.float32),
            jax.ShapeDtypeStruct((K * K,), jnp.float32),
        ),
        mesh=mesh,
        compiler_params=pltpu.CompilerParams(needs_layout_passes=False),
        scratch_types=[
            pltpu.VMEM((KPAD,), jnp.int32),
            pltpu.VMEM((BLK_PER_TILE * 16,), jnp.int32),
            pltpu.VMEM((RB * N,), jnp.float32),
            pltpu.VMEM((RB * KPAD,), jnp.float32),
            pltpu.VMEM((RB * F,), jnp.float32),
            pltpu.SemaphoreType.DMA,
            pltpu.SemaphoreType.DMA,
        ],
    )
    def gather_k(x_hbm, a_hbm, idxc_hbm, idxr_hbm, xpool_hbm, apool_hbm,
                 idxc_v, idxr_v, rows_v, out_v, xrows_v, sem_a, sem_x):
        wid = lax.axis_index("s") * 2 + lax.axis_index("c")
        pltpu.sync_copy(idxc_hbm, idxc_v)
        pltpu.sync_copy(
            idxr_hbm.at[pl.ds(wid * BLK_PER_TILE * 16, BLK_PER_TILE * 16)],
            idxr_v)
        lane = lax.iota(jnp.int32, 16)
        int_min = jnp.int32(-2147483648)

        def blk(bi, carry):
            b = wid * BLK_PER_TILE + bi

            @pl.when(b < NBLK)
            def _():
                rvec = idxr_v[pl.ds(pl.multiple_of(bi * 16, 16), 16)]
                copies = []
                for j in range(RB):
                    rid = jnp.max(jnp.where(lane == j, rvec, int_min))
                    copies.append(pltpu.async_copy(
                        a_hbm.at[pl.ds(pl.multiple_of(rid * N, 8), N)],
                        rows_v.at[pl.ds(j * N, N)], sem_a))
                    copies.append(pltpu.async_copy(
                        x_hbm.at[rid], xrows_v.at[pl.ds(j * F, F)], sem_x))
                for cp in copies:
                    cp.wait()

                def grp(g, c2):
                    off = pl.multiple_of(g * 16, 16)
                    cidx = idxc_v[pl.ds(off, 16)]
                    for j in range(RB):
                        vals = plsc.load_gather(rows_v, [cidx + j * N])
                        out_v[pl.ds(pl.multiple_of(j * KPAD, 16) + off, 16)] = vals
                    return c2

                lax.fori_loop(0, KPAD // 16, grp, 0)
                r0 = b * RB
                for j in range(RB):
                    pltpu.sync_copy(out_v.at[pl.ds(j * KPAD, K)],
                                    apool_hbm.at[pl.ds((r0 + j) * K, K)])
                pltpu.sync_copy(xrows_v, xpool_hbm.at[pl.ds(r0 * F, RB * F)])

            return carry

        lax.fori_loop(0, BLK_PER_TILE, blk, 0)

    return gather_k


_gather_cached = functools.cache(_make_gather)


def kernel(X, A, W):
    x_pad = jnp.pad(X, ((0, NPAD - N), (0, 0)))
    scol = _scores(x_pad, W)
    s2d = scol.reshape(NROWS2D, LANES)
    ranks = _ranks(scol, s2d)
    idx_full = _scatter_cached()(ranks.reshape(NPAD))
    idx = lax.slice(idx_full, (0,), (K,))
    idxc = lax.slice(idx_full, (0,), (KPAD,))
    idxr = jnp.pad(idx.reshape(NBLK, RB),
                   ((0, NTILES * BLK_PER_TILE - NBLK), (0, 16 - RB))).reshape(-1)
    x_flat, a_flat = _gather_cached()(X, A.reshape(-1), idxc, idxr)
    return (x_flat.reshape(K, F), a_flat.reshape(K, K), idx)


# speedup vs baseline: 2.1319x; 1.2224x over previous
import functools

import jax
import jax.numpy as jnp
from jax import lax
from jax.experimental import pallas as pl
from jax.experimental.pallas import tpu as pltpu
from jax.experimental.pallas import tpu_sc as plsc

N = 10000
F = 128
K = 5000
LANES = 128
NPAD = 10240
NROWS2D = NPAD // LANES
IBLK = 1024
NB = NPAD // IBLK
KPAD = 5008
RB = 4
NBLK = K // RB
NTILES = 32
BLK_PER_TILE = -(-NBLK // NTILES)
SCAT_PER_TILE = NPAD // NTILES


def _scores_body(x_ref, w_ref, o_ref):
    s = jnp.dot(x_ref[...], w_ref[...])
    rid = lax.broadcasted_iota(jnp.int32, (NPAD, 1), 0)
    o_ref[...] = jnp.where(rid < N, s, -jnp.inf)


def _scores(x_pad, w):
    return pl.pallas_call(
        _scores_body,
        out_shape=jax.ShapeDtypeStruct((NPAD, 1), jnp.float32),
    )(x_pad, w)


def _ranks_body(scol_ref, s2d_ref, o_ref):
    ib = pl.program_id(0)
    si = scol_ref[...]
    d0 = ib * (IBLK // LANES)

    def before(rb, acc):
        sj = s2d_ref[pl.ds(rb, 1), :]
        return acc + jnp.sum(jnp.where(sj >= si, 1.0, 0.0), axis=1,
                             keepdims=True)

    def after(rb, acc):
        sj = s2d_ref[pl.ds(rb, 1), :]
        return acc + jnp.sum(jnp.where(sj > si, 1.0, 0.0), axis=1,
                             keepdims=True)

    ii = ib * IBLK + lax.broadcasted_iota(jnp.int32, (IBLK, 1), 0)

    def diag(rb, acc):
        sj = s2d_ref[pl.ds(rb, 1), :]
        jj = rb * LANES + lax.broadcasted_iota(jnp.int32, (1, LANES), 1)
        cmp = (sj > si) | ((sj == si) & (jj < ii))
        return acc + jnp.sum(jnp.where(cmp, 1.0, 0.0), axis=1, keepdims=True)

    acc = jnp.zeros((IBLK, 1), jnp.float32)
    acc = lax.fori_loop(0, d0, before, acc)
    acc = lax.fori_loop(d0, d0 + IBLK // LANES, diag, acc)
    acc = lax.fori_loop(d0 + IBLK // LANES, NROWS2D, after, acc)
    o_ref[...] = acc.astype(jnp.int32)


def _ranks(scol, s2d):
    return pl.pallas_call(
        _ranks_body,
        grid=(NB,),
        in_specs=[
            pl.BlockSpec((IBLK, 1), lambda i: (i, 0)),
            pl.BlockSpec((NROWS2D, LANES), lambda i: (0, 0)),
        ],
        out_specs=pl.BlockSpec((IBLK, 1), lambda i: (i, 0)),
        out_shape=jax.ShapeDtypeStruct((NPAD, 1), jnp.int32),
    )(scol, s2d)


def _make_scatter():
    mesh = plsc.VectorSubcoreMesh(core_axis_name="c", subcore_axis_name="s")

    @functools.partial(
        pl.kernel,
        out_type=jax.ShapeDtypeStruct((NPAD,), jnp.int32),
        mesh=mesh,
        compiler_params=pltpu.CompilerParams(needs_layout_passes=False),
        scratch_types=[
            pltpu.VMEM((SCAT_PER_TILE,), jnp.int32),
            pltpu.VMEM((SCAT_PER_TILE,), jnp.int32),
            pltpu.SemaphoreType.DMA,
        ],
    )
    def scatter_k(ranks_hbm, out_hbm, ranks_v, vals_v, sem):
        wid = lax.axis_index("s") * 2 + lax.axis_index("c")
        base = wid * SCAT_PER_TILE
        pltpu.sync_copy(ranks_hbm.at[pl.ds(base, SCAT_PER_TILE)], ranks_v)
        for g in range(SCAT_PER_TILE // 16):
            vals_v[pl.ds(g * 16, 16)] = base + g * 16 + lax.iota(jnp.int32, 16)
        handles = []
        for g in range(SCAT_PER_TILE // 16):
            rvec = ranks_v[pl.ds(g * 16, 16)]
            handles.append(pltpu.async_copy(
                vals_v.at[pl.ds(g * 16, 16)], out_hbm.at[rvec], sem))
        for h in handles:
            h.wait()

    return scatter_k


_scatter_cached = functools.cache(_make_scatter)


def _make_gather():
    mesh = plsc.VectorSubcoreMesh(core_axis_name="c", subcore_axis_name="s")

    @functools.partial(
        pl.kernel,
        out_type=jax.ShapeDtypeStruct((K * K,), jnp.float32),
        mesh=mesh,
        compiler_params=pltpu.CompilerParams(needs_layout_passes=False),
        scratch_types=[
            pltpu.VMEM((KPAD,), jnp.int32),
            pltpu.VMEM((BLK_PER_TILE * 16,), jnp.int32),
            pltpu.VMEM((2 * RB * N,), jnp.float32),
            pltpu.VMEM((2 * RB * KPAD,), jnp.float32),
            pltpu.SemaphoreType.DMA,
            pltpu.SemaphoreType.DMA,
            pltpu.SemaphoreType.DMA,
            pltpu.SemaphoreType.DMA,
        ],
    )
    def gather_k(a_hbm, idxc_hbm, idxr_hbm, apool_hbm,
                 idxc_v, idxr_v, rows_v, out_v, sem_a0, sem_a1, sem_o0,
                 sem_o1):
        wid = lax.axis_index("s") * 2 + lax.axis_index("c")
        sems_a = (sem_a0, sem_a1)
        sems_o = (sem_o0, sem_o1)
        base = wid * BLK_PER_TILE
        pltpu.sync_copy(idxc_hbm, idxc_v)
        pltpu.sync_copy(
            idxr_hbm.at[pl.ds(base * 16, BLK_PER_TILE * 16)], idxr_v)
        lane = lax.iota(jnp.int32, 16)
        int_min = jnp.int32(-2147483648)

        def issue_rows(bi, buf):
            rvec = idxr_v[pl.ds(pl.multiple_of(bi * 16, 16), 16)]
            for j in range(RB):
                rid = jnp.max(jnp.where(lane == j, rvec, int_min))
                pltpu.async_copy(
                    a_hbm.at[pl.ds(pl.multiple_of(rid * N, 8), N)],
                    rows_v.at[pl.ds((buf * RB + j) * N, N)], sems_a[buf])

        def drain_rows(buf):
            for j in range(RB):
                pltpu.make_async_copy(
                    a_hbm.at[pl.ds(0, N)],
                    rows_v.at[pl.ds((buf * RB + j) * N, N)],
                    sems_a[buf]).wait()

        def drain_out(buf):
            for j in range(RB):
                pltpu.make_async_copy(
                    out_v.at[pl.ds((buf * RB + j) * KPAD, K)],
                    apool_hbm.at[pl.ds(0, K)], sems_o[buf]).wait()

        @pl.when(base < NBLK)
        def _prime():
            issue_rows(0, 0)

        def pair(p, carry):
            for sub in (0, 1):
                bi = 2 * p + sub
                b = base + bi

                @pl.when(b < NBLK)
                def _():
                    @pl.when((b + 1 < NBLK) & (bi + 1 < BLK_PER_TILE))
                    def _issue_next():
                        issue_rows(bi + 1, 1 - sub)

                    drain_rows(sub)

                    @pl.when(bi >= 2)
                    def _drain_prev_out():
                        drain_out(sub)

                    def grp(g, c2):
                        off = pl.multiple_of(g * 16, 16)
                        cidx = idxc_v[pl.ds(off, 16)]
                        for j in range(RB):
                            vals = plsc.load_gather(
                                rows_v, [cidx + (sub * RB + j) * N])
                            out_v[pl.ds(pl.multiple_of(
                                (sub * RB + j) * KPAD, 16) + off, 16)] = vals
                        return c2

                    lax.fori_loop(0, KPAD // 16, grp, 0)
                    r0 = b * RB
                    for j in range(RB):
                        pltpu.async_copy(
                            out_v.at[pl.ds((sub * RB + j) * KPAD, K)],
                            apool_hbm.at[pl.ds((r0 + j) * K, K)],
                            sems_o[sub])
            return carry

        lax.fori_loop(0, BLK_PER_TILE // 2, pair, 0)
        for sub in (0, 1):
            @pl.when(base + sub < NBLK)
            def _final_drain(sub=sub):
                drain_out(sub)

    return gather_k


_gather_cached = functools.cache(_make_gather)


KX = 5120
XPT = KX // NTILES


def _make_xgather():
    mesh = plsc.VectorSubcoreMesh(core_axis_name="c", subcore_axis_name="s")

    @functools.partial(
        pl.kernel,
        out_type=jax.ShapeDtypeStruct((KX, F), jnp.float32),
        mesh=mesh,
        compiler_params=pltpu.CompilerParams(needs_layout_passes=False),
        scratch_types=[
            pltpu.VMEM((XPT,), jnp.int32),
            pltpu.VMEM((XPT, F), jnp.float32),
            pltpu.SemaphoreType.DMA,
        ],
    )
    def xgather_k(x_hbm, idxx_hbm, xpool_hbm, idxx_v, xrows_v, sem):
        wid = lax.axis_index("s") * 2 + lax.axis_index("c")
        pltpu.sync_copy(idxx_hbm.at[pl.ds(wid * XPT, XPT)], idxx_v)
        h0 = pltpu.async_copy(x_hbm.at[idxx_v.at[pl.ds(0, 80)]],
                              xrows_v.at[pl.ds(0, 80)], sem)
        h1 = pltpu.async_copy(x_hbm.at[idxx_v.at[pl.ds(80, 80)]],
                              xrows_v.at[pl.ds(80, 80)], sem)
        h0.wait()
        h1.wait()
        pltpu.sync_copy(xrows_v, xpool_hbm.at[pl.ds(wid * XPT, XPT)])

    return xgather_k


_xgather_cached = functools.cache(_make_xgather)


def kernel(X, A, W):
    x_pad = jnp.pad(X, ((0, NPAD - N), (0, 0)))
    scol = _scores(x_pad, W)
    s2d = scol.reshape(NROWS2D, LANES)
    ranks = _ranks(scol, s2d)
    idx_full = _scatter_cached()(ranks.reshape(NPAD))
    idx = lax.slice(idx_full, (0,), (K,))
    idxc = lax.slice(idx_full, (0,), (KPAD,))
    idxr = jnp.pad(idx.reshape(NBLK, RB),
                   ((0, NTILES * BLK_PER_TILE - NBLK), (0, 16 - RB))).reshape(-1)
    a_flat = _gather_cached()(A.reshape(-1), idxc, idxr)
    idxx = lax.slice(idx_full, (0,), (KX,))
    xpool = _xgather_cached()(X, idxx)
    x_pooled = lax.slice(xpool, (0, 0), (K, F))
    return (x_pooled, a_flat.reshape(K, K), idx)


# speedup vs baseline: 3.2543x; 1.5265x over previous
import functools

import jax
import jax.numpy as jnp
from jax import lax
from jax.experimental import pallas as pl
from jax.experimental.pallas import tpu as pltpu
from jax.experimental.pallas import tpu_sc as plsc

N = 10000
F = 128
K = 5000
LANES = 128
NPAD = 10240
NROWS2D = NPAD // LANES
IBLK = 1024
NB = NPAD // IBLK
KPAD = 5008
RB = 4
NBLK = K // RB
NTILES = 32
BLK_PER_TILE = -(-NBLK // NTILES)
SCAT_PER_TILE = NPAD // NTILES


def _scores_body(x_ref, w_ref, o_ref):
    s = jnp.dot(x_ref[...], w_ref[...])
    rid = lax.broadcasted_iota(jnp.int32, (NPAD, 1), 0)
    o_ref[...] = jnp.where(rid < N, s, -jnp.inf)


def _scores(x_pad, w):
    return pl.pallas_call(
        _scores_body,
        out_shape=jax.ShapeDtypeStruct((NPAD, 1), jnp.float32),
    )(x_pad, w)


def _ranks_body(scol_ref, s2d_ref, o_ref, sib_ref):
    ib = pl.program_id(0)
    sib_ref[...] = jnp.broadcast_to(scol_ref[...], (IBLK, LANES))
    jlt = (lax.broadcasted_iota(jnp.int32, (LANES, LANES), 1)
           < lax.broadcasted_iota(jnp.int32, (LANES, LANES), 0))

    for isub in range(IBLK // LANES):
        si = sib_ref[pl.ds(isub * LANES, LANES), :]
        ri = ib * (IBLK // LANES) + isub

        def before(rb, acc):
            sj = s2d_ref[pl.ds(rb, 1), :]
            return acc + jnp.where(sj >= si, 1.0, 0.0)

        def after(rb, acc):
            sj = s2d_ref[pl.ds(rb, 1), :]
            return acc + jnp.where(sj > si, 1.0, 0.0)

        acc = jnp.zeros((LANES, LANES), jnp.float32)
        acc = lax.fori_loop(0, ri, before, acc)
        sj = s2d_ref[pl.ds(ri, 1), :]
        acc = acc + jnp.where((sj > si) | ((sj == si) & jlt), 1.0, 0.0)
        acc = lax.fori_loop(ri + 1, NROWS2D, after, acc)
        o_ref[pl.ds(isub * LANES, LANES), :] = (
            jnp.sum(acc, axis=1, keepdims=True).astype(jnp.int32))


def _ranks(scol, s2d):
    return pl.pallas_call(
        _ranks_body,
        grid=(NB,),
        in_specs=[
            pl.BlockSpec((IBLK, 1), lambda i: (i, 0)),
            pl.BlockSpec((NROWS2D, LANES), lambda i: (0, 0)),
        ],
        out_specs=pl.BlockSpec((IBLK, 1), lambda i: (i, 0)),
        out_shape=jax.ShapeDtypeStruct((NPAD, 1), jnp.int32),
        scratch_shapes=[pltpu.VMEM((IBLK, LANES), jnp.float32)],
    )(scol, s2d)


def _make_scatter():
    mesh = plsc.VectorSubcoreMesh(core_axis_name="c", subcore_axis_name="s")

    @functools.partial(
        pl.kernel,
        out_type=jax.ShapeDtypeStruct((NPAD,), jnp.int32),
        mesh=mesh,
        compiler_params=pltpu.CompilerParams(needs_layout_passes=False),
        scratch_types=[
            pltpu.VMEM((80,), jnp.int32),
            pltpu.VMEM((80,), jnp.int32),
            pltpu.VMEM((80,), jnp.int32),
            pltpu.VMEM((80,), jnp.int32),
            pltpu.VMEM((SCAT_PER_TILE,), jnp.int32),
            pltpu.SemaphoreType.DMA,
        ],
    )
    def scatter_k(ranks_hbm, out_hbm, r0, r1, r2, r3, vals_v, sem):
        wid = lax.axis_index("s") * 2 + lax.axis_index("c")
        base = wid * SCAT_PER_TILE
        rrefs = (r0, r1, r2, r3)
        for j in range(4):
            pltpu.sync_copy(ranks_hbm.at[pl.ds(base + j * 80, 80)], rrefs[j])
        for g in range(SCAT_PER_TILE // 16):
            vals_v[pl.ds(g * 16, 16)] = base + g * 16 + lax.iota(jnp.int32, 16)
        handles = []
        for j in range(4):
            handles.append(pltpu.async_copy(
                vals_v.at[pl.ds(j * 80, 80)], out_hbm.at[rrefs[j]], sem))
        for h in handles:
            h.wait()

    return scatter_k


_scatter_cached = functools.cache(_make_scatter)


def _make_gather():
    mesh = plsc.VectorSubcoreMesh(core_axis_name="c", subcore_axis_name="s")

    @functools.partial(
        pl.kernel,
        out_type=jax.ShapeDtypeStruct((K * K,), jnp.float32),
        mesh=mesh,
        compiler_params=pltpu.CompilerParams(needs_layout_passes=False),
        scratch_types=[
            pltpu.VMEM((KPAD,), jnp.int32),
            pltpu.VMEM((BLK_PER_TILE * 16,), jnp.int32),
            pltpu.VMEM((2 * RB * N,), jnp.float32),
            pltpu.VMEM((2 * RB * KPAD,), jnp.float32),
            pltpu.SemaphoreType.DMA,
            pltpu.SemaphoreType.DMA,
            pltpu.SemaphoreType.DMA,
            pltpu.SemaphoreType.DMA,
        ],
    )
    def gather_k(a_hbm, idxc_hbm, idxr_hbm, apool_hbm,
                 idxc_v, idxr_v, rows_v, out_v, sem_a0, sem_a1, sem_o0,
                 sem_o1):
        wid = lax.axis_index("s") * 2 + lax.axis_index("c")
        sems_a = (sem_a0, sem_a1)
        sems_o = (sem_o0, sem_o1)
        base = wid * BLK_PER_TILE
        pltpu.sync_copy(idxc_hbm, idxc_v)
        pltpu.sync_copy(
            idxr_hbm.at[pl.ds(base * 16, BLK_PER_TILE * 16)], idxr_v)
        lane = lax.iota(jnp.int32, 16)
        int_min = jnp.int32(-2147483648)

        def issue_rows(bi, buf):
            rvec = idxr_v[pl.ds(pl.multiple_of(bi * 16, 16), 16)]
            for j in range(RB):
                rid = jnp.max(jnp.where(lane == j, rvec, int_min))
                pltpu.async_copy(
                    a_hbm.at[pl.ds(pl.multiple_of(rid * N, 8), N)],
                    rows_v.at[pl.ds((buf * RB + j) * N, N)], sems_a[buf])

        def drain_rows(buf):
            for j in range(RB):
                pltpu.make_async_copy(
                    a_hbm.at[pl.ds(0, N)],
                    rows_v.at[pl.ds((buf * RB + j) * N, N)],
                    sems_a[buf]).wait()

        def drain_out(buf):
            for j in range(RB):
                pltpu.make_async_copy(
                    out_v.at[pl.ds((buf * RB + j) * KPAD, K)],
                    apool_hbm.at[pl.ds(0, K)], sems_o[buf]).wait()

        @pl.when(base < NBLK)
        def _prime():
            issue_rows(0, 0)

        def pair(p, carry):
            for sub in (0, 1):
                bi = 2 * p + sub
                b = base + bi

                @pl.when(b < NBLK)
                def _():
                    @pl.when((b + 1 < NBLK) & (bi + 1 < BLK_PER_TILE))
                    def _issue_next():
                        issue_rows(bi + 1, 1 - sub)

                    drain_rows(sub)

                    @pl.when(bi >= 2)
                    def _drain_prev_out():
                        drain_out(sub)

                    @functools.partial(
                        plsc.parallel_loop, 0, KPAD // 16, unroll=4)
                    def grp(g):
                        off = pl.multiple_of(g * 16, 16)
                        cidx = idxc_v[pl.ds(off, 16)]
                        for j in range(RB):
                            vals = plsc.load_gather(
                                rows_v, [cidx + (sub * RB + j) * N])
                            out_v[pl.ds(pl.multiple_of(
                                (sub * RB + j) * KPAD, 16) + off, 16)] = vals
                    r0 = b * RB
                    for j in range(RB):
                        pltpu.async_copy(
                            out_v.at[pl.ds((sub * RB + j) * KPAD, K)],
                            apool_hbm.at[pl.ds((r0 + j) * K, K)],
                            sems_o[sub])
            return carry

        lax.fori_loop(0, BLK_PER_TILE // 2, pair, 0)
        for sub in (0, 1):
            @pl.when(base + sub < NBLK)
            def _final_drain(sub=sub):
                drain_out(sub)

    return gather_k


_gather_cached = functools.cache(_make_gather)


KX = 5120
XPT = KX // NTILES


def _make_xgather():
    mesh = plsc.VectorSubcoreMesh(core_axis_name="c", subcore_axis_name="s")

    @functools.partial(
        pl.kernel,
        out_type=jax.ShapeDtypeStruct((KX, F), jnp.float32),
        mesh=mesh,
        compiler_params=pltpu.CompilerParams(needs_layout_passes=False),
        scratch_types=[
            pltpu.VMEM((XPT,), jnp.int32),
            pltpu.VMEM((XPT, F), jnp.float32),
            pltpu.SemaphoreType.DMA,
        ],
    )
    def xgather_k(x_hbm, idxx_hbm, xpool_hbm, idxx_v, xrows_v, sem):
        wid = lax.axis_index("s") * 2 + lax.axis_index("c")
        pltpu.sync_copy(idxx_hbm.at[pl.ds(wid * XPT, XPT)], idxx_v)
        h0 = pltpu.async_copy(x_hbm.at[idxx_v.at[pl.ds(0, 80)]],
                              xrows_v.at[pl.ds(0, 80)], sem)
        h1 = pltpu.async_copy(x_hbm.at[idxx_v.at[pl.ds(80, 80)]],
                              xrows_v.at[pl.ds(80, 80)], sem)
        h0.wait()
        h1.wait()
        pltpu.sync_copy(xrows_v, xpool_hbm.at[pl.ds(wid * XPT, XPT)])

    return xgather_k


_xgather_cached = functools.cache(_make_xgather)


def kernel(X, A, W):
    x_pad = jnp.pad(X, ((0, NPAD - N), (0, 0)))
    scol = _scores(x_pad, W)
    s2d = scol.reshape(NROWS2D, LANES)
    ranks = _ranks(scol, s2d)
    idx_full = _scatter_cached()(ranks.reshape(NPAD))
    idx = lax.slice(idx_full, (0,), (K,))
    idxc = lax.slice(idx_full, (0,), (KPAD,))
    idxr = jnp.pad(idx.reshape(NBLK, RB),
                   ((0, NTILES * BLK_PER_TILE - NBLK), (0, 16 - RB))).reshape(-1)
    a_flat = _gather_cached()(A.reshape(-1), idxc, idxr)
    idxx = lax.slice(idx_full, (0,), (KX,))
    xpool = _xgather_cached()(X, idxx)
    x_pooled = lax.slice(xpool, (0, 0), (K, F))
    return (x_pooled, a_flat.reshape(K, K), idx)
